# baseline (speedup 1.0000x reference)
import functools

import jax
import jax.numpy as jnp
from jax import lax
from jax.experimental import pallas as pl
from jax.experimental.pallas import tpu as pltpu
from jax.experimental.pallas import tpu_sc as plsc

_EPS = 1.0
_LAMB = 0.5
_N_NODES = 50000
_N_EDGES = 1600000

_NW = 32
_E_PER_W = _N_EDGES // _NW
_ECH = 10000
_ECHUNKS = _E_PER_W // _ECH
_NODE_WORKERS = 25
_NCH = _N_NODES // _NODE_WORKERS


def _nll_partial(mu_b, s2_b, val_b, nvec, acc):
    lanes = lax.iota(jnp.int32, 16)
    lo = lanes < 8
    idx0 = (lanes & 7) * 2
    idx1 = idx0 + 1

    def body(j, a):
        o = j * 16
        mu = mu_b[pl.ds(o, 16)]
        s2 = s2_b[pl.ds(o, 16)]
        pa = val_b[pl.ds(2 * o, 16)]
        pb = val_b[pl.ds(2 * o + 16, 16)]
        v0 = jnp.where(lo, pa.at[idx0].get(mode="promise_in_bounds"),
                       pb.at[idx0].get(mode="promise_in_bounds"))
        v1 = jnp.where(lo, pa.at[idx1].get(mode="promise_in_bounds"),
                       pb.at[idx1].get(mode="promise_in_bounds"))
        x = s2 + _EPS
        t = s2 / (s2 + 2.0)
        t2 = t * t
        lg = t * (2.0 + t2 * (2.0 / 3.0 + t2 * (2.0 / 5.0 + t2 * (2.0 / 7.0 + t2 * (2.0 / 9.0)))))
        d = v0 - mu
        return a + (0.5 * lg + d * d / x) * v1

    return lax.fori_loop(0, nvec, body, acc)


def _sc_body(n_mu, n_s2, e_mu, e_s2, nkey, nval, ekey, eval_,
             out_node, out_edge,
             ekey_b, emu_b, es2_b, eval_b,
             nkey_b, nmu_b, ns2_b, nval_b,
             stage_b, sem0, sem1, sem2):
    cid = lax.axis_index("c")
    sid = lax.axis_index("s")
    wid = sid * 2 + cid

    def echunk(c, acc):
        base = pl.multiple_of(wid * _E_PER_W + c * _ECH, 8)
        cpv = pltpu.async_copy(eval_.at[pl.ds(2 * base, 2 * _ECH)], eval_b, sem2)
        pltpu.sync_copy(ekey.at[pl.ds(base, _ECH)], ekey_b)
        cp0 = pltpu.async_copy(e_mu.at[ekey_b], emu_b, sem0)
        cp1 = pltpu.async_copy(e_s2.at[ekey_b], es2_b, sem1)
        cpv.wait()
        cp0.wait()
        cp1.wait()
        return _nll_partial(emu_b, es2_b, eval_b, _ECH // 16, acc)

    eacc = lax.fori_loop(0, _ECHUNKS, echunk, jnp.zeros((16,), jnp.float32))
    stage_b[...] = eacc
    pltpu.sync_copy(stage_b, out_edge.at[wid])

    stage_b[...] = jnp.zeros((16,), jnp.float32)

    @pl.when(wid < _NODE_WORKERS)
    def _():
        base = pl.multiple_of(wid * _NCH, 8)
        cpv = pltpu.async_copy(nval.at[pl.ds(2 * base, 2 * _NCH)], nval_b, sem2)
        pltpu.sync_copy(nkey.at[pl.ds(base, _NCH)], nkey_b)
        cp0 = pltpu.async_copy(n_mu.at[nkey_b], nmu_b, sem0)
        cp1 = pltpu.async_copy(n_s2.at[nkey_b], ns2_b, sem1)
        cpv.wait()
        cp0.wait()
        cp1.wait()
        nacc = _nll_partial(nmu_b, ns2_b, nval_b, _NCH // 16,
                            jnp.zeros((16,), jnp.float32))
        stage_b[...] = nacc

    pltpu.sync_copy(stage_b, out_node.at[wid])


_sc_kernel = pl.kernel(
    _sc_body,
    out_type=(jax.ShapeDtypeStruct((_NW, 16), jnp.float32),
              jax.ShapeDtypeStruct((_NW, 16), jnp.float32)),
    mesh=plsc.VectorSubcoreMesh(core_axis_name="c", subcore_axis_name="s"),
    scratch_types=[
        pltpu.VMEM((_ECH,), jnp.int32),
        pltpu.VMEM((_ECH,), jnp.float32),
        pltpu.VMEM((_ECH,), jnp.float32),
        pltpu.VMEM((2 * _ECH,), jnp.float32),
        pltpu.VMEM((_NCH,), jnp.int32),
        pltpu.VMEM((_NCH,), jnp.float32),
        pltpu.VMEM((_NCH,), jnp.float32),
        pltpu.VMEM((2 * _NCH,), jnp.float32),
        pltpu.VMEM((16,), jnp.float32),
        pltpu.SemaphoreType.DMA,
        pltpu.SemaphoreType.DMA,
        pltpu.SemaphoreType.DMA,
    ],
)


def _combine_body(np_ref, ep_ref, on_ref, oe_ref, ot_ref):
    n = jnp.sum(np_ref[...]) * (1.0 / _N_NODES)
    e = jnp.sum(ep_ref[...]) * (1.0 / _N_EDGES)
    on_ref[0, 0] = n
    oe_ref[0, 0] = e
    ot_ref[0, 0] = n * _LAMB + e * (1.0 - _LAMB)


_combine = pl.pallas_call(
    _combine_body,
    out_shape=(jax.ShapeDtypeStruct((1, 1), jnp.float32),
               jax.ShapeDtypeStruct((1, 1), jnp.float32),
               jax.ShapeDtypeStruct((1, 1), jnp.float32)),
    out_specs=(pl.BlockSpec(memory_space=pltpu.SMEM),
               pl.BlockSpec(memory_space=pltpu.SMEM),
               pl.BlockSpec(memory_space=pltpu.SMEM)),
)


def kernel(n_mu, n_sigma2, e_mu, e_sigma2, batch_node_key, batch_node_value,
           batch_edge_key, batch_edge_value):
    node_pp, edge_pp = _sc_kernel(
        n_mu, n_sigma2, e_mu, e_sigma2,
        batch_node_key.astype(jnp.int32), batch_node_value.reshape(-1),
        batch_edge_key.astype(jnp.int32), batch_edge_value.reshape(-1))
    on, oe, ot = _combine(node_pp, edge_pp)
    return (on[0, 0], oe[0, 0], ot[0, 0])


# speedup vs baseline: 12.7325x; 12.7325x over previous
import jax
import jax.numpy as jnp
from jax import lax
from jax.experimental import pallas as pl
from jax.experimental.pallas import tpu as pltpu
from jax.experimental.pallas import tpu_sc as plsc

_EPS = 1.0
_LAMB = 0.5
_N_NODES = 50000
_N_EDGES = 1600000

_NW = 32
_E_PER_W = _N_EDGES // _NW
_GCH = 25000
_NODE_WORKERS = 25
_NCH = _N_NODES // _NODE_WORKERS


def _gather_body(n_mu, n_s2, e_mu, e_s2, nkey, ekey,
                 gnmu, gns2, gemu, ges2,
                 key_b, mu0_b, s20_b, mu1_b, s21_b, semg, semw):
    cid = lax.axis_index("c")
    sid = lax.axis_index("s")
    wid = sid * 2 + cid

    @pl.when(wid < _NODE_WORKERS)
    def _():
        nb = pl.multiple_of(wid * _NCH, 8)
        kv = key_b.at[pl.ds(0, _NCH)]
        pltpu.sync_copy(nkey.at[pl.ds(nb, _NCH)], kv)
        c0 = pltpu.async_copy(n_mu.at[kv], mu0_b.at[pl.ds(0, _NCH)], semg)
        c1 = pltpu.async_copy(n_s2.at[kv], s20_b.at[pl.ds(0, _NCH)], semg)
        c0.wait()
        c1.wait()
        pltpu.sync_copy(mu0_b.at[pl.ds(0, _NCH)], gnmu.at[pl.ds(nb, _NCH)])
        pltpu.sync_copy(s20_b.at[pl.ds(0, _NCH)], gns2.at[pl.ds(nb, _NCH)])

    eb0 = pl.multiple_of(wid * _E_PER_W, 8)
    eb1 = pl.multiple_of(wid * _E_PER_W + _GCH, 8)

    pltpu.sync_copy(ekey.at[pl.ds(eb0, _GCH)], key_b)
    g0m = pltpu.async_copy(e_mu.at[key_b], mu0_b, semg)
    g0s = pltpu.async_copy(e_s2.at[key_b], s20_b, semg)
    g0m.wait()
    g0s.wait()
    w0m = pltpu.async_copy(mu0_b, gemu.at[pl.ds(eb0, _GCH)], semw)
    w0s = pltpu.async_copy(s20_b, ges2.at[pl.ds(eb0, _GCH)], semw)

    pltpu.sync_copy(ekey.at[pl.ds(eb1, _GCH)], key_b)
    g1m = pltpu.async_copy(e_mu.at[key_b], mu1_b, semg)
    g1s = pltpu.async_copy(e_s2.at[key_b], s21_b, semg)
    g1m.wait()
    g1s.wait()
    w1m = pltpu.async_copy(mu1_b, gemu.at[pl.ds(eb1, _GCH)], semw)
    w1s = pltpu.async_copy(s21_b, ges2.at[pl.ds(eb1, _GCH)], semw)

    w0m.wait()
    w0s.wait()
    w1m.wait()
    w1s.wait()


_sc_gather = pl.kernel(
    _gather_body,
    out_type=(jax.ShapeDtypeStruct((_N_NODES,), jnp.float32),
              jax.ShapeDtypeStruct((_N_NODES,), jnp.float32),
              jax.ShapeDtypeStruct((_N_EDGES,), jnp.float32),
              jax.ShapeDtypeStruct((_N_EDGES,), jnp.float32)),
    mesh=plsc.VectorSubcoreMesh(core_axis_name="c", subcore_axis_name="s"),
    scratch_types=[
        pltpu.VMEM((_GCH,), jnp.int32),
        pltpu.VMEM((_GCH,), jnp.float32),
        pltpu.VMEM((_GCH,), jnp.float32),
        pltpu.VMEM((_GCH,), jnp.float32),
        pltpu.VMEM((_GCH,), jnp.float32),
        pltpu.SemaphoreType.DMA,
        pltpu.SemaphoreType.DMA,
    ],
)

_BE = 131072
_GE = -(-_N_EDGES // _BE)


def _nll_tc_body(gnmu, gns2, nv0, nv1, gemu, ges2, ev0, ev1,
                 on_ref, oe_ref, ot_ref, accn, acce):
    pid = pl.program_id(0)

    @pl.when(pid == 0)
    def _():
        x = gns2[...] + _EPS
        d = nv0[...] - gnmu[...]
        accn[0] = jnp.sum((0.5 * jnp.log(x) + d * d / x) * nv1[...])
        acce[0] = 0.0

    x = ges2[...] + _EPS
    d = ev0[...] - gemu[...]
    term = (0.5 * jnp.log(x) + d * d / x) * ev1[...]
    valid = pid * _BE + lax.iota(jnp.int32, _BE) < _N_EDGES
    acce[0] += jnp.sum(jnp.where(valid, term, 0.0))

    @pl.when(pid == _GE - 1)
    def _():
        n = accn[0] * (1.0 / _N_NODES)
        e = acce[0] * (1.0 / _N_EDGES)
        on_ref[0, 0] = n
        oe_ref[0, 0] = e
        ot_ref[0, 0] = n * _LAMB + e * (1.0 - _LAMB)


_nll_tc = pl.pallas_call(
    _nll_tc_body,
    grid=(_GE,),
    in_specs=[
        pl.BlockSpec((_N_NODES,), lambda i: (0,)),
        pl.BlockSpec((_N_NODES,), lambda i: (0,)),
        pl.BlockSpec((_N_NODES,), lambda i: (0,)),
        pl.BlockSpec((_N_NODES,), lambda i: (0,)),
        pl.BlockSpec((_BE,), lambda i: (i,)),
        pl.BlockSpec((_BE,), lambda i: (i,)),
        pl.BlockSpec((_BE,), lambda i: (i,)),
        pl.BlockSpec((_BE,), lambda i: (i,)),
    ],
    out_shape=(jax.ShapeDtypeStruct((1, 1), jnp.float32),
               jax.ShapeDtypeStruct((1, 1), jnp.float32),
               jax.ShapeDtypeStruct((1, 1), jnp.float32)),
    out_specs=(pl.BlockSpec(memory_space=pltpu.SMEM),
               pl.BlockSpec(memory_space=pltpu.SMEM),
               pl.BlockSpec(memory_space=pltpu.SMEM)),
    scratch_shapes=[pltpu.SMEM((1,), jnp.float32),
                    pltpu.SMEM((1,), jnp.float32)],
)


def kernel(n_mu, n_sigma2, e_mu, e_sigma2, batch_node_key, batch_node_value,
           batch_edge_key, batch_edge_value):
    gnmu, gns2, gemu, ges2 = _sc_gather(
        n_mu, n_sigma2, e_mu, e_sigma2,
        batch_node_key.astype(jnp.int32), batch_edge_key.astype(jnp.int32))
    on, oe, ot = _nll_tc(
        gnmu, gns2, batch_node_value[:, 0], batch_node_value[:, 1],
        gemu, ges2, batch_edge_value[:, 0], batch_edge_value[:, 1])
    return (on[0, 0], oe[0, 0], ot[0, 0])
